# baseline (speedup 1.0000x reference)
import functools

import jax
import jax.numpy as jnp
from jax import lax
from jax.experimental import pallas as pl
from jax.experimental.pallas import tpu as pltpu
from jax.experimental.pallas import tpu_sc as plsc

N_NODES = 10000
IN_DIM = 2048
PN = 10240
E_RAW = 160000
EP = 163840
NC, NS = 2, 16
GP = EP // (NC * NS * 128)
GP0, GP1 = 24, 56
GPMX = max(GP0, GP1)
RPT = PN // NS
PAD_NODE = PN - 1
BM = 256
NB = PN // BM


def _sc_mesh():
    return plsc.VectorSubcoreMesh(
        core_axis_name="c", subcore_axis_name="s", num_cores=NC, num_subcores=NS
    )


PR = PN // 128


def _split_idx(i16):
    return [lax.shift_right_logical(i16, 7), jnp.bitwise_and(i16, 127)]


@functools.partial(
    pl.kernel,
    out_type=jax.ShapeDtypeStruct((NC, NS, PR, 128), jnp.float32),
    mesh=_sc_mesh(),
    compiler_params=pltpu.CompilerParams(
        needs_layout_passes=False, use_tc_tiling_on_sc=False),
    scratch_types=[
        pltpu.VMEM((GPMX, 128), jnp.int32),
        pltpu.VMEM((PR, 128), jnp.float32),
    ],
    name="sc_degree",
)
def _sc_degree(dst_hbm, out_hbm, dst_v, part_v):
    cid = lax.axis_index("c")
    sid = lax.axis_index("s")
    cnt = jnp.where(cid == 0, GP0, GP1)
    base = jnp.where(cid == 0, sid * GP0, NS * GP0 + sid * GP1)
    pltpu.sync_copy(dst_hbm.at[pl.ds(base, GPMX)], dst_v)
    zeros = jnp.zeros((16,), jnp.float32)

    @pl.loop(0, PR)
    def _(i):
        @pl.loop(0, 8)
        def _(j):
            part_v[i, pl.ds(j * 16, 16)] = zeros

    ones = jnp.ones((16,), jnp.float32)

    @pl.loop(0, cnt)
    def _(g):
        @pl.loop(0, 8)
        def _(j):
            d16 = dst_v[g, pl.ds(j * 16, 16)]
            plsc.addupdate_scatter(part_v, _split_idx(d16), ones)

    pltpu.sync_copy(part_v, out_hbm.at[cid, sid])


def _make_sc_scatter(C, Fc, dtype=jnp.float32):
    @functools.partial(
        pl.kernel,
        out_type=jax.ShapeDtypeStruct((NC, C, PN, Fc), dtype),
        mesh=_sc_mesh(),
        compiler_params=pltpu.CompilerParams(
            needs_layout_passes=False, use_tc_tiling_on_sc=False),
        scratch_types=[
            pltpu.VMEM((GPMX, 128), jnp.int32),
            pltpu.VMEM((GPMX, 128), jnp.int32),
            pltpu.VMEM((128, Fc), dtype),
            pltpu.VMEM((128, Fc), dtype),
            pltpu.VMEM_SHARED((PN, Fc), dtype),
            pltpu.SemaphoreType.DMA,
            pltpu.SemaphoreType.DMA,
            pltpu.SemaphoreType.DMA,
            pltpu.SemaphoreType.DMA,
        ],
        name=f"sc_scatter_{C}x{Fc}",
    )
    def k(h_hbm, src_hbm, dst_hbm, z_hbm, out_hbm,
          src_v, dst_v, r0, r1, acc_sh, sg0, sg1, ss0, ss1):
        cid = lax.axis_index("c")
        sid = lax.axis_index("s")
        cnt = jnp.where(cid == 0, GP0, GP1)
        base = jnp.where(cid == 0, sid * GP0, NS * GP0 + sid * GP1)
        pltpu.sync_copy(src_hbm.at[pl.ds(base, GPMX)], src_v)
        pltpu.sync_copy(dst_hbm.at[pl.ds(base, GPMX)], dst_v)
        for c in range(C):
            for z in range(RPT // 128):
                pltpu.sync_copy(
                    z_hbm, acc_sh.at[pl.ds(sid * RPT + z * 128, 128)]
                )
            plsc.subcore_barrier()
            pltpu.async_copy(h_hbm.at[c].at[src_v.at[0]], r0, sg0)

            @pl.loop(0, cnt // 2)
            def _(t):
                g = t * 2
                pltpu.make_async_copy(
                    h_hbm.at[c].at[src_v.at[g]], r0, sg0).wait()
                pltpu.async_copy(h_hbm.at[c].at[src_v.at[g + 1]], r1, sg1)
                pltpu.async_copy(r0, acc_sh.at[dst_v.at[g]], ss0, add=True)
                pltpu.make_async_copy(
                    h_hbm.at[c].at[src_v.at[g + 1]], r1, sg1).wait()
                pltpu.make_async_copy(r0, acc_sh.at[dst_v.at[g]], ss0).wait()

                @pl.when(t < cnt // 2 - 1)
                def _():
                    pltpu.async_copy(
                        h_hbm.at[c].at[src_v.at[g + 2]], r0, sg0)

                pltpu.async_copy(r1, acc_sh.at[dst_v.at[g + 1]], ss1, add=True)
                pltpu.make_async_copy(
                    r1, acc_sh.at[dst_v.at[g + 1]], ss1).wait()

            plsc.subcore_barrier()
            pltpu.sync_copy(
                acc_sh.at[pl.ds(sid * RPT, RPT)],
                out_hbm.at[cid, c, pl.ds(sid * RPT, RPT)],
            )
            if c < C - 1:
                plsc.subcore_barrier()

    return k


@functools.partial(
    pl.kernel,
    out_type=jax.ShapeDtypeStruct((NC, PN, 48), jnp.float32),
    mesh=_sc_mesh(),
    compiler_params=pltpu.CompilerParams(
        needs_layout_passes=False, use_tc_tiling_on_sc=False),
    scratch_types=[
        pltpu.VMEM((GPMX, 128), jnp.int32),
        pltpu.VMEM((GPMX, 128), jnp.int32),
        pltpu.VMEM((PR, 128), jnp.float32),
        pltpu.VMEM((PR, 128), jnp.float32),
        pltpu.VMEM((128,), jnp.float32),
        pltpu.VMEM((128, 48), jnp.float32),
        pltpu.VMEM((128, 48), jnp.float32),
        pltpu.VMEM_SHARED((PN, 48), jnp.float32),
        pltpu.SemaphoreType.DMA,
        pltpu.SemaphoreType.DMA,
        pltpu.SemaphoreType.DMA,
        pltpu.SemaphoreType.DMA,
    ],
    name="sc_gat",
)
def _sc_gat(hg_hbm, es_hbm, ed_hbm, src_hbm, dst_hbm, z_hbm,
            num_hbm,
            src_v, dst_v, es_v, ed_v, w_v, r0, r1, acc_sh,
            sg0, sg1, ss0, ss1):
    cid = lax.axis_index("c")
    sid = lax.axis_index("s")
    cnt = jnp.where(cid == 0, GP0, GP1)
    base = jnp.where(cid == 0, sid * GP0, NS * GP0 + sid * GP1)
    pltpu.sync_copy(src_hbm.at[pl.ds(base, GPMX)], src_v)
    pltpu.sync_copy(dst_hbm.at[pl.ds(base, GPMX)], dst_v)
    pltpu.sync_copy(es_hbm, es_v)
    pltpu.sync_copy(ed_hbm, ed_v)
    for z in range(RPT // 128):
        pltpu.sync_copy(z_hbm, acc_sh.at[pl.ds(sid * RPT + z * 128, 128)])
    plsc.subcore_barrier()

    onehot = (lax.iota(jnp.int32, 16) == 0).astype(jnp.float32)

    def _weigh(g, rows_v):
        @pl.loop(0, 8)
        def _(j):
            s16 = src_v[g, pl.ds(j * 16, 16)]
            d16 = dst_v[g, pl.ds(j * 16, 16)]
            z16 = (plsc.load_gather(es_v, _split_idx(s16))
                   + plsc.load_gather(ed_v, _split_idx(d16)))
            w16 = jnp.exp(jnp.maximum(z16, 0.2 * z16))
            w_v[pl.ds(j * 16, 16)] = w16

        @pl.loop(0, 8)
        def _(r):
            w16 = w_v[pl.ds(r * 16, 16)]
            for k in range(16):
                w_s = w16[k]
                row = r * 16 + k
                rows_v[row, pl.ds(0, 16)] = rows_v[row, pl.ds(0, 16)] * w_s
                rows_v[row, pl.ds(16, 16)] = rows_v[row, pl.ds(16, 16)] * w_s
                rows_v[row, pl.ds(32, 16)] = onehot * w_s

    pltpu.async_copy(hg_hbm.at[src_v.at[0]], r0, sg0)

    @pl.loop(0, cnt // 2)
    def _(t):
        g = t * 2
        pltpu.make_async_copy(hg_hbm.at[src_v.at[g]], r0, sg0).wait()
        pltpu.async_copy(hg_hbm.at[src_v.at[g + 1]], r1, sg1)
        _weigh(g, r0)
        pltpu.async_copy(r0, acc_sh.at[dst_v.at[g]], ss0, add=True)
        pltpu.make_async_copy(hg_hbm.at[src_v.at[g + 1]], r1, sg1).wait()
        pltpu.make_async_copy(r0, acc_sh.at[dst_v.at[g]], ss0).wait()

        @pl.when(t < cnt // 2 - 1)
        def _():
            pltpu.async_copy(hg_hbm.at[src_v.at[g + 2]], r0, sg0)

        _weigh(g + 1, r1)
        pltpu.async_copy(r1, acc_sh.at[dst_v.at[g + 1]], ss1, add=True)
        pltpu.make_async_copy(r1, acc_sh.at[dst_v.at[g + 1]], ss1).wait()

    plsc.subcore_barrier()
    pltpu.sync_copy(
        acc_sh.at[pl.ds(sid * RPT, RPT)],
        num_hbm.at[cid, pl.ds(sid * RPT, RPT)],
    )


def _rdeg_body(p_ref, out_ref):
    deg = jnp.sum(p_ref[...], axis=(0, 1)) + 1.0
    out_ref[...] = lax.rsqrt(deg)


def _rden_body(p_ref, out_ref):
    out_ref[...] = jnp.sum(p_ref[...], axis=(0, 1))


def _reduce_parts(parts, is_deg):
    out2d = pl.pallas_call(
        _rdeg_body if is_deg else _rden_body,
        out_shape=jax.ShapeDtypeStruct((PR, 128), jnp.float32),
    )(parts)
    return out2d.reshape(PN, 1)


def _mm1_body(x_ref, w_ref, dinv_ref, out_ref, outb_ref):
    h = jnp.dot(x_ref[...], w_ref[...], preferred_element_type=jnp.float32)
    hs = h * dinv_ref[...]
    for c in range(2):
        blk = hs[:, c * 256:(c + 1) * 256]
        out_ref[c, :, :] = blk
        outb_ref[c, :, :] = blk.astype(jnp.bfloat16)


def _mm1(xp, W1, dinv):
    return pl.pallas_call(
        _mm1_body,
        grid=(NB,),
        in_specs=[
            pl.BlockSpec((BM, IN_DIM), lambda i: (i, 0)),
            pl.BlockSpec((IN_DIM, 512), lambda i: (0, 0)),
            pl.BlockSpec((BM, 1), lambda i: (i, 0)),
        ],
        out_specs=[
            pl.BlockSpec((2, BM, 256), lambda i: (0, i, 0)),
            pl.BlockSpec((2, BM, 256), lambda i: (0, i, 0)),
        ],
        out_shape=[
            jax.ShapeDtypeStruct((2, PN, 256), jnp.float32),
            jax.ShapeDtypeStruct((2, PN, 256), jnp.bfloat16),
        ],
    )(xp, W1, dinv)


def _post_mm_body(C_in, F_out, s_ref, hs_ref, dinv_ref, b_ref, g_ref,
                  be_ref, w_ref, out_ref, outb_ref):
    dv = dinv_ref[...]
    sagg = (s_ref[0].astype(jnp.float32) + s_ref[1].astype(jnp.float32))
    y = (sagg + hs_ref[...]) * dv[None]
    y = y + b_ref[...][:, None, :]
    mu = jnp.mean(y, axis=(0, 2), keepdims=True)
    var = jnp.mean((y - mu) ** 2, axis=(0, 2), keepdims=True)
    yn = (y - mu) * lax.rsqrt(var + 1e-5)
    yn = yn * g_ref[...][:, None, :] + be_ref[...][:, None, :]
    xr = jnp.maximum(yn, 0.0)
    acc = jnp.zeros((BM, F_out), jnp.float32)
    for c in range(C_in):
        acc = acc + jnp.dot(xr[c], w_ref[c], preferred_element_type=jnp.float32)
    res = acc * dv
    out_ref[0, :, :] = res
    outb_ref[0, :, :] = res.astype(jnp.bfloat16)


def _post_mm(C_in, Fc_in, F_out, S, hs, dinv, b, g, be, Wc):
    body = functools.partial(_post_mm_body, C_in, F_out)
    return pl.pallas_call(
        body,
        grid=(NB,),
        in_specs=[
            pl.BlockSpec((NC, C_in, BM, Fc_in), lambda i: (0, 0, i, 0)),
            pl.BlockSpec((C_in, BM, Fc_in), lambda i: (0, i, 0)),
            pl.BlockSpec((BM, 1), lambda i: (i, 0)),
            pl.BlockSpec((C_in, Fc_in), lambda i: (0, 0)),
            pl.BlockSpec((C_in, Fc_in), lambda i: (0, 0)),
            pl.BlockSpec((C_in, Fc_in), lambda i: (0, 0)),
            pl.BlockSpec((C_in, Fc_in, F_out), lambda i: (0, 0, 0)),
        ],
        out_specs=[
            pl.BlockSpec((1, BM, F_out), lambda i: (0, i, 0)),
            pl.BlockSpec((1, BM, F_out), lambda i: (0, i, 0)),
        ],
        out_shape=[
            jax.ShapeDtypeStruct((1, PN, F_out), jnp.float32),
            jax.ShapeDtypeStruct((1, PN, F_out), jnp.bfloat16),
        ],
    )(S, hs, dinv, b, g, be, Wc)


def _gatprep_body(s_ref, hs_ref, dinv_ref, b_ref, g_ref, be_ref,
                  wg_ref, as_ref, ad_ref, hg_ref, es_ref, ed_ref):
    dv = dinv_ref[...]
    y = (s_ref[0, 0].astype(jnp.float32) + s_ref[1, 0].astype(jnp.float32)
         + hs_ref[0]) * dv + b_ref[...]
    mu = jnp.mean(y, axis=1, keepdims=True)
    var = jnp.mean((y - mu) ** 2, axis=1, keepdims=True)
    yn = (y - mu) * lax.rsqrt(var + 1e-5) * g_ref[...] + be_ref[...]
    x4 = jnp.maximum(yn, 0.0)
    hg = jnp.dot(x4, wg_ref[...], preferred_element_type=jnp.float32)
    hg_ref[...] = jnp.concatenate(
        [hg, jnp.zeros((BM, 16), jnp.float32)], axis=1)
    es_ref[...] = jnp.dot(hg, as_ref[...], preferred_element_type=jnp.float32)
    ed_ref[...] = jnp.dot(hg, ad_ref[...], preferred_element_type=jnp.float32)


def _gatprep(S3, h3s, dinv, b3, g3, be3, Wg, a_s, a_d):
    return pl.pallas_call(
        _gatprep_body,
        grid=(NB,),
        in_specs=[
            pl.BlockSpec((NC, 1, BM, 32), lambda i: (0, 0, i, 0)),
            pl.BlockSpec((1, BM, 32), lambda i: (0, i, 0)),
            pl.BlockSpec((BM, 1), lambda i: (i, 0)),
            pl.BlockSpec((1, 32), lambda i: (0, 0)),
            pl.BlockSpec((1, 32), lambda i: (0, 0)),
            pl.BlockSpec((1, 32), lambda i: (0, 0)),
            pl.BlockSpec((32, 32), lambda i: (0, 0)),
            pl.BlockSpec((32, 1), lambda i: (0, 0)),
            pl.BlockSpec((32, 1), lambda i: (0, 0)),
        ],
        out_specs=[
            pl.BlockSpec((BM, 48), lambda i: (i, 0)),
            pl.BlockSpec((BM, 1), lambda i: (i, 0)),
            pl.BlockSpec((BM, 1), lambda i: (i, 0)),
        ],
        out_shape=[
            jax.ShapeDtypeStruct((PN, 48), jnp.float32),
            jax.ShapeDtypeStruct((PN, 1), jnp.float32),
            jax.ShapeDtypeStruct((PN, 1), jnp.float32),
        ],
    )(S3, h3s, dinv, b3, g3, be3, Wg, a_s, a_d)


def _gatfinal_body(num_ref, hg_ref, es_ref, ed_ref, bg_ref,
                   wf_ref, bf_ref, acc_ref, out_ref):
    i = pl.program_id(0)
    z = es_ref[...] + ed_ref[...]
    wself = jnp.exp(jnp.maximum(z, 0.2 * z))
    hg = hg_ref[...][:, :32]
    nd = num_ref[0] + num_ref[1]
    num = nd[:, :32] + wself * hg
    den = nd[:, 32:33] + wself + 1e-16
    hgat = jnp.maximum(num / den + bg_ref[...], 0.0)
    rows = lax.broadcasted_iota(jnp.int32, (BM, 1), 0) + i * BM
    hgat = jnp.where(rows < N_NODES, hgat, 0.0)
    part = jnp.sum(hgat, axis=0, keepdims=True)

    @pl.when(i == 0)
    def _():
        acc_ref[...] = jnp.zeros_like(acc_ref)

    acc_ref[...] += part

    @pl.when(i == NB - 1)
    def _():
        pooled = acc_ref[...] * (1.0 / N_NODES)
        out_ref[...] = (
            jnp.dot(pooled, wf_ref[...], preferred_element_type=jnp.float32)
            + bf_ref[...]
        )


def _gatfinal(num, hg, es, ed, bg, Wf, bf):
    _, out = pl.pallas_call(
        _gatfinal_body,
        grid=(NB,),
        in_specs=[
            pl.BlockSpec((NC, BM, 48), lambda i: (0, i, 0)),
            pl.BlockSpec((BM, 48), lambda i: (i, 0)),
            pl.BlockSpec((BM, 1), lambda i: (i, 0)),
            pl.BlockSpec((BM, 1), lambda i: (i, 0)),
            pl.BlockSpec((1, 32), lambda i: (0, 0)),
            pl.BlockSpec((32, 2), lambda i: (0, 0)),
            pl.BlockSpec((1, 2), lambda i: (0, 0)),
        ],
        out_specs=[
            pl.BlockSpec((1, 32), lambda i: (0, 0)),
            pl.BlockSpec((1, 2), lambda i: (0, 0)),
        ],
        out_shape=[
            jax.ShapeDtypeStruct((1, 32), jnp.float32),
            jax.ShapeDtypeStruct((1, 2), jnp.float32),
        ],
    )(num, hg, es, ed, bg, Wf, bf)
    return out


_scatter_2x256 = _make_sc_scatter(2, 256, jnp.bfloat16)
_scatter_1x128 = _make_sc_scatter(1, 128, jnp.bfloat16)
_scatter_1x32 = _make_sc_scatter(1, 32, jnp.bfloat16)


def kernel(x, edge_index, W1, b1, g1, be1, W2, b2, g2, be2, W3, b3, g3, be3,
           Wg, a_s, a_d, bg, Wf, bf):
    f32 = jnp.float32
    xp = jnp.pad(x, ((0, PN - N_NODES), (0, 0)))
    pad = jnp.full((EP - E_RAW,), PAD_NODE, jnp.int32)
    src2d = jnp.concatenate([edge_index[0], pad]).reshape(EP // 128, 128)
    dst2d = jnp.concatenate([edge_index[1], pad]).reshape(EP // 128, 128)
    bf16 = jnp.bfloat16
    z256b = jnp.zeros((128, 256), bf16)
    z128b = jnp.zeros((128, 128), bf16)
    z32b = jnp.zeros((128, 32), bf16)
    z48 = jnp.zeros((128, 48), f32)

    degp = _sc_degree(dst2d)
    dinv = _reduce_parts(degp, is_deg=True)

    h1s, h1sb = _mm1(xp, W1, dinv)
    S1 = _scatter_2x256(h1sb, src2d, dst2d, z256b)
    h2s, h2sb = _post_mm(2, 256, 128, S1, h1s, dinv,
                         b1.reshape(2, 256), g1.reshape(2, 256),
                         be1.reshape(2, 256),
                         W2.reshape(2, 256, 128))
    S2 = _scatter_1x128(h2sb, src2d, dst2d, z128b)
    h3s, h3sb = _post_mm(1, 128, 32, S2, h2s, dinv,
                         b2.reshape(1, 128), g2.reshape(1, 128),
                         be2.reshape(1, 128),
                         W3.reshape(1, 128, 32))
    S3 = _scatter_1x32(h3sb, src2d, dst2d, z32b)
    hg, es, ed = _gatprep(S3, h3s, dinv,
                          b3.reshape(1, 32), g3.reshape(1, 32),
                          be3.reshape(1, 32), Wg,
                          a_s.reshape(32, 1), a_d.reshape(32, 1))
    num = _sc_gat(hg, es.reshape(PR, 128), ed.reshape(PR, 128),
                  src2d, dst2d, z48)
    return _gatfinal(num, hg, es, ed, bg.reshape(1, 32), Wf,
                     bf.reshape(1, 2))


# speedup vs baseline: 1.1307x; 1.1307x over previous
import functools

import jax
import jax.numpy as jnp
from jax import lax
from jax.experimental import pallas as pl
from jax.experimental.pallas import tpu as pltpu
from jax.experimental.pallas import tpu_sc as plsc

N_NODES = 10000
IN_DIM = 2048
PN = 10240
E_RAW = 160000
EP = 163840
NC, NS = 2, 16
GP = EP // (NC * NS * 128)
GP0, GP1 = 56, 24
GPMX = max(GP0, GP1)
RPT = PN // NS
PAD_NODE = PN - 1
BM = 256
NB = PN // BM


def _sc_mesh():
    return plsc.VectorSubcoreMesh(
        core_axis_name="c", subcore_axis_name="s", num_cores=NC, num_subcores=NS
    )


PR = PN // 128


def _split_idx(i16):
    return [lax.shift_right_logical(i16, 7), jnp.bitwise_and(i16, 127)]


@functools.partial(
    pl.kernel,
    out_type=jax.ShapeDtypeStruct((NC, NS, PR, 128), jnp.float32),
    mesh=_sc_mesh(),
    compiler_params=pltpu.CompilerParams(
        needs_layout_passes=False, use_tc_tiling_on_sc=False),
    scratch_types=[
        pltpu.VMEM((GPMX, 128), jnp.int32),
        pltpu.VMEM((PR, 128), jnp.float32),
    ],
    name="sc_degree",
)
def _sc_degree(dst_hbm, out_hbm, dst_v, part_v):
    cid = lax.axis_index("c")
    sid = lax.axis_index("s")
    cnt = jnp.where(cid == 0, GP0, GP1)
    base = jnp.where(cid == 0, sid * GP0, NS * GP0 + sid * GP1)
    pltpu.sync_copy(dst_hbm.at[pl.ds(base, GPMX)], dst_v)
    zeros = jnp.zeros((16,), jnp.float32)

    @pl.loop(0, PR)
    def _(i):
        @pl.loop(0, 8)
        def _(j):
            part_v[i, pl.ds(j * 16, 16)] = zeros

    ones = jnp.ones((16,), jnp.float32)

    @pl.loop(0, cnt)
    def _(g):
        @pl.loop(0, 8)
        def _(j):
            d16 = dst_v[g, pl.ds(j * 16, 16)]
            plsc.addupdate_scatter(part_v, _split_idx(d16), ones)

    pltpu.sync_copy(part_v, out_hbm.at[cid, sid])


def _make_sc_scatter(C, Fc, dtype=jnp.float32):
    @functools.partial(
        pl.kernel,
        out_type=jax.ShapeDtypeStruct((NC, C, PN, Fc), dtype),
        mesh=_sc_mesh(),
        compiler_params=pltpu.CompilerParams(
            needs_layout_passes=False, use_tc_tiling_on_sc=False),
        scratch_types=[
            pltpu.VMEM((GPMX, 128), jnp.int32),
            pltpu.VMEM((GPMX, 128), jnp.int32),
            pltpu.VMEM((128, Fc), dtype),
            pltpu.VMEM((128, Fc), dtype),
            pltpu.VMEM_SHARED((PN, Fc), dtype),
            pltpu.SemaphoreType.DMA,
            pltpu.SemaphoreType.DMA,
            pltpu.SemaphoreType.DMA,
            pltpu.SemaphoreType.DMA,
        ],
        name=f"sc_scatter_{C}x{Fc}",
    )
    def k(h_hbm, src_hbm, dst_hbm, z_hbm, out_hbm,
          src_v, dst_v, r0, r1, acc_sh, sg0, sg1, ss0, ss1):
        cid = lax.axis_index("c")
        sid = lax.axis_index("s")
        cnt = jnp.where(cid == 0, GP0, GP1)
        base = jnp.where(cid == 0, sid * GP0, NS * GP0 + sid * GP1)
        pltpu.sync_copy(src_hbm.at[pl.ds(base, GPMX)], src_v)
        pltpu.sync_copy(dst_hbm.at[pl.ds(base, GPMX)], dst_v)
        for c in range(C):
            for z in range(RPT // 128):
                pltpu.sync_copy(
                    z_hbm, acc_sh.at[pl.ds(sid * RPT + z * 128, 128)]
                )
            plsc.subcore_barrier()
            pltpu.async_copy(h_hbm.at[c].at[src_v.at[0]], r0, sg0)

            @pl.loop(0, cnt // 2)
            def _(t):
                g = t * 2
                pltpu.make_async_copy(
                    h_hbm.at[c].at[src_v.at[g]], r0, sg0).wait()
                pltpu.async_copy(h_hbm.at[c].at[src_v.at[g + 1]], r1, sg1)
                pltpu.async_copy(r0, acc_sh.at[dst_v.at[g]], ss0, add=True)
                pltpu.make_async_copy(
                    h_hbm.at[c].at[src_v.at[g + 1]], r1, sg1).wait()
                pltpu.make_async_copy(r0, acc_sh.at[dst_v.at[g]], ss0).wait()

                @pl.when(t < cnt // 2 - 1)
                def _():
                    pltpu.async_copy(
                        h_hbm.at[c].at[src_v.at[g + 2]], r0, sg0)

                pltpu.async_copy(r1, acc_sh.at[dst_v.at[g + 1]], ss1, add=True)
                pltpu.make_async_copy(
                    r1, acc_sh.at[dst_v.at[g + 1]], ss1).wait()

            plsc.subcore_barrier()
            pltpu.sync_copy(
                acc_sh.at[pl.ds(sid * RPT, RPT)],
                out_hbm.at[cid, c, pl.ds(sid * RPT, RPT)],
            )
            if c < C - 1:
                plsc.subcore_barrier()

    return k


@functools.partial(
    pl.kernel,
    out_type=jax.ShapeDtypeStruct((NC, PN, 48), jnp.float32),
    mesh=_sc_mesh(),
    compiler_params=pltpu.CompilerParams(
        needs_layout_passes=False, use_tc_tiling_on_sc=False),
    scratch_types=[
        pltpu.VMEM((GPMX, 128), jnp.int32),
        pltpu.VMEM((GPMX, 128), jnp.int32),
        pltpu.VMEM((PR, 128), jnp.float32),
        pltpu.VMEM((PR, 128), jnp.float32),
        pltpu.VMEM((128,), jnp.float32),
        pltpu.VMEM((128, 48), jnp.float32),
        pltpu.VMEM((128, 48), jnp.float32),
        pltpu.VMEM_SHARED((PN, 48), jnp.float32),
        pltpu.SemaphoreType.DMA,
        pltpu.SemaphoreType.DMA,
        pltpu.SemaphoreType.DMA,
        pltpu.SemaphoreType.DMA,
    ],
    name="sc_gat",
)
def _sc_gat(hg_hbm, es_hbm, ed_hbm, src_hbm, dst_hbm, z_hbm,
            num_hbm,
            src_v, dst_v, es_v, ed_v, w_v, r0, r1, acc_sh,
            sg0, sg1, ss0, ss1):
    cid = lax.axis_index("c")
    sid = lax.axis_index("s")
    cnt = jnp.where(cid == 0, GP0, GP1)
    base = jnp.where(cid == 0, sid * GP0, NS * GP0 + sid * GP1)
    pltpu.sync_copy(src_hbm.at[pl.ds(base, GPMX)], src_v)
    pltpu.sync_copy(dst_hbm.at[pl.ds(base, GPMX)], dst_v)
    pltpu.sync_copy(es_hbm, es_v)
    pltpu.sync_copy(ed_hbm, ed_v)
    for z in range(RPT // 128):
        pltpu.sync_copy(z_hbm, acc_sh.at[pl.ds(sid * RPT + z * 128, 128)])
    plsc.subcore_barrier()

    onehot = (lax.iota(jnp.int32, 16) == 0).astype(jnp.float32)

    def _weigh(g, rows_v):
        @pl.loop(0, 8)
        def _(j):
            s16 = src_v[g, pl.ds(j * 16, 16)]
            d16 = dst_v[g, pl.ds(j * 16, 16)]
            z16 = (plsc.load_gather(es_v, _split_idx(s16))
                   + plsc.load_gather(ed_v, _split_idx(d16)))
            w16 = jnp.exp(jnp.maximum(z16, 0.2 * z16))
            w_v[pl.ds(j * 16, 16)] = w16

        @pl.loop(0, 8)
        def _(r):
            w16 = w_v[pl.ds(r * 16, 16)]
            for k in range(16):
                w_s = w16[k]
                row = r * 16 + k
                rows_v[row, pl.ds(0, 16)] = rows_v[row, pl.ds(0, 16)] * w_s
                rows_v[row, pl.ds(16, 16)] = rows_v[row, pl.ds(16, 16)] * w_s
                rows_v[row, pl.ds(32, 16)] = onehot * w_s

    pltpu.async_copy(hg_hbm.at[src_v.at[0]], r0, sg0)

    @pl.loop(0, cnt // 2)
    def _(t):
        g = t * 2
        pltpu.make_async_copy(hg_hbm.at[src_v.at[g]], r0, sg0).wait()
        pltpu.async_copy(hg_hbm.at[src_v.at[g + 1]], r1, sg1)
        _weigh(g, r0)
        pltpu.async_copy(r0, acc_sh.at[dst_v.at[g]], ss0, add=True)
        pltpu.make_async_copy(hg_hbm.at[src_v.at[g + 1]], r1, sg1).wait()
        pltpu.make_async_copy(r0, acc_sh.at[dst_v.at[g]], ss0).wait()

        @pl.when(t < cnt // 2 - 1)
        def _():
            pltpu.async_copy(hg_hbm.at[src_v.at[g + 2]], r0, sg0)

        _weigh(g + 1, r1)
        pltpu.async_copy(r1, acc_sh.at[dst_v.at[g + 1]], ss1, add=True)
        pltpu.make_async_copy(r1, acc_sh.at[dst_v.at[g + 1]], ss1).wait()

    plsc.subcore_barrier()
    pltpu.sync_copy(
        acc_sh.at[pl.ds(sid * RPT, RPT)],
        num_hbm.at[cid, pl.ds(sid * RPT, RPT)],
    )


def _rdeg_body(p_ref, out_ref):
    deg = jnp.sum(p_ref[...], axis=(0, 1)) + 1.0
    out_ref[...] = lax.rsqrt(deg)


def _rden_body(p_ref, out_ref):
    out_ref[...] = jnp.sum(p_ref[...], axis=(0, 1))


def _reduce_parts(parts, is_deg):
    out2d = pl.pallas_call(
        _rdeg_body if is_deg else _rden_body,
        out_shape=jax.ShapeDtypeStruct((PR, 128), jnp.float32),
    )(parts)
    return out2d.reshape(PN, 1)


def _mm1_body(x_ref, w_ref, dinv_ref, out_ref, outb_ref):
    h = jnp.dot(x_ref[...], w_ref[...], preferred_element_type=jnp.float32)
    hs = h * dinv_ref[...]
    for c in range(2):
        blk = hs[:, c * 256:(c + 1) * 256]
        out_ref[c, :, :] = blk
        outb_ref[c, :, :] = blk.astype(jnp.bfloat16)


def _mm1(xp, W1, dinv):
    return pl.pallas_call(
        _mm1_body,
        grid=(NB,),
        in_specs=[
            pl.BlockSpec((BM, IN_DIM), lambda i: (i, 0)),
            pl.BlockSpec((IN_DIM, 512), lambda i: (0, 0)),
            pl.BlockSpec((BM, 1), lambda i: (i, 0)),
        ],
        out_specs=[
            pl.BlockSpec((2, BM, 256), lambda i: (0, i, 0)),
            pl.BlockSpec((2, BM, 256), lambda i: (0, i, 0)),
        ],
        out_shape=[
            jax.ShapeDtypeStruct((2, PN, 256), jnp.float32),
            jax.ShapeDtypeStruct((2, PN, 256), jnp.bfloat16),
        ],
    )(xp, W1, dinv)


def _post_mm_body(C_in, F_out, s_ref, hs_ref, dinv_ref, b_ref, g_ref,
                  be_ref, w_ref, out_ref, outb_ref):
    dv = dinv_ref[...]
    sagg = (s_ref[0].astype(jnp.float32) + s_ref[1].astype(jnp.float32))
    y = (sagg + hs_ref[...]) * dv[None]
    y = y + b_ref[...][:, None, :]
    mu = jnp.mean(y, axis=(0, 2), keepdims=True)
    var = jnp.mean((y - mu) ** 2, axis=(0, 2), keepdims=True)
    yn = (y - mu) * lax.rsqrt(var + 1e-5)
    yn = yn * g_ref[...][:, None, :] + be_ref[...][:, None, :]
    xr = jnp.maximum(yn, 0.0)
    acc = jnp.zeros((BM, F_out), jnp.float32)
    for c in range(C_in):
        acc = acc + jnp.dot(xr[c], w_ref[c], preferred_element_type=jnp.float32)
    res = acc * dv
    out_ref[0, :, :] = res
    outb_ref[0, :, :] = res.astype(jnp.bfloat16)


def _post_mm(C_in, Fc_in, F_out, S, hs, dinv, b, g, be, Wc):
    body = functools.partial(_post_mm_body, C_in, F_out)
    return pl.pallas_call(
        body,
        grid=(NB,),
        in_specs=[
            pl.BlockSpec((NC, C_in, BM, Fc_in), lambda i: (0, 0, i, 0)),
            pl.BlockSpec((C_in, BM, Fc_in), lambda i: (0, i, 0)),
            pl.BlockSpec((BM, 1), lambda i: (i, 0)),
            pl.BlockSpec((C_in, Fc_in), lambda i: (0, 0)),
            pl.BlockSpec((C_in, Fc_in), lambda i: (0, 0)),
            pl.BlockSpec((C_in, Fc_in), lambda i: (0, 0)),
            pl.BlockSpec((C_in, Fc_in, F_out), lambda i: (0, 0, 0)),
        ],
        out_specs=[
            pl.BlockSpec((1, BM, F_out), lambda i: (0, i, 0)),
            pl.BlockSpec((1, BM, F_out), lambda i: (0, i, 0)),
        ],
        out_shape=[
            jax.ShapeDtypeStruct((1, PN, F_out), jnp.float32),
            jax.ShapeDtypeStruct((1, PN, F_out), jnp.bfloat16),
        ],
    )(S, hs, dinv, b, g, be, Wc)


def _gatprep_body(s_ref, hs_ref, dinv_ref, b_ref, g_ref, be_ref,
                  wg_ref, as_ref, ad_ref, hg_ref, es_ref, ed_ref):
    dv = dinv_ref[...]
    y = (s_ref[0, 0].astype(jnp.float32) + s_ref[1, 0].astype(jnp.float32)
         + hs_ref[0]) * dv + b_ref[...]
    mu = jnp.mean(y, axis=1, keepdims=True)
    var = jnp.mean((y - mu) ** 2, axis=1, keepdims=True)
    yn = (y - mu) * lax.rsqrt(var + 1e-5) * g_ref[...] + be_ref[...]
    x4 = jnp.maximum(yn, 0.0)
    hg = jnp.dot(x4, wg_ref[...], preferred_element_type=jnp.float32)
    hg_ref[...] = jnp.concatenate(
        [hg, jnp.zeros((BM, 16), jnp.float32)], axis=1)
    es_ref[...] = jnp.dot(hg, as_ref[...], preferred_element_type=jnp.float32)
    ed_ref[...] = jnp.dot(hg, ad_ref[...], preferred_element_type=jnp.float32)


def _gatprep(S3, h3s, dinv, b3, g3, be3, Wg, a_s, a_d):
    return pl.pallas_call(
        _gatprep_body,
        grid=(NB,),
        in_specs=[
            pl.BlockSpec((NC, 1, BM, 32), lambda i: (0, 0, i, 0)),
            pl.BlockSpec((1, BM, 32), lambda i: (0, i, 0)),
            pl.BlockSpec((BM, 1), lambda i: (i, 0)),
            pl.BlockSpec((1, 32), lambda i: (0, 0)),
            pl.BlockSpec((1, 32), lambda i: (0, 0)),
            pl.BlockSpec((1, 32), lambda i: (0, 0)),
            pl.BlockSpec((32, 32), lambda i: (0, 0)),
            pl.BlockSpec((32, 1), lambda i: (0, 0)),
            pl.BlockSpec((32, 1), lambda i: (0, 0)),
        ],
        out_specs=[
            pl.BlockSpec((BM, 48), lambda i: (i, 0)),
            pl.BlockSpec((BM, 1), lambda i: (i, 0)),
            pl.BlockSpec((BM, 1), lambda i: (i, 0)),
        ],
        out_shape=[
            jax.ShapeDtypeStruct((PN, 48), jnp.float32),
            jax.ShapeDtypeStruct((PN, 1), jnp.float32),
            jax.ShapeDtypeStruct((PN, 1), jnp.float32),
        ],
    )(S3, h3s, dinv, b3, g3, be3, Wg, a_s, a_d)


def _gatfinal_body(num_ref, hg_ref, es_ref, ed_ref, bg_ref,
                   wf_ref, bf_ref, acc_ref, out_ref):
    i = pl.program_id(0)
    z = es_ref[...] + ed_ref[...]
    wself = jnp.exp(jnp.maximum(z, 0.2 * z))
    hg = hg_ref[...][:, :32]
    nd = num_ref[0] + num_ref[1]
    num = nd[:, :32] + wself * hg
    den = nd[:, 32:33] + wself + 1e-16
    hgat = jnp.maximum(num / den + bg_ref[...], 0.0)
    rows = lax.broadcasted_iota(jnp.int32, (BM, 1), 0) + i * BM
    hgat = jnp.where(rows < N_NODES, hgat, 0.0)
    part = jnp.sum(hgat, axis=0, keepdims=True)

    @pl.when(i == 0)
    def _():
        acc_ref[...] = jnp.zeros_like(acc_ref)

    acc_ref[...] += part

    @pl.when(i == NB - 1)
    def _():
        pooled = acc_ref[...] * (1.0 / N_NODES)
        out_ref[...] = (
            jnp.dot(pooled, wf_ref[...], preferred_element_type=jnp.float32)
            + bf_ref[...]
        )


def _gatfinal(num, hg, es, ed, bg, Wf, bf):
    _, out = pl.pallas_call(
        _gatfinal_body,
        grid=(NB,),
        in_specs=[
            pl.BlockSpec((NC, BM, 48), lambda i: (0, i, 0)),
            pl.BlockSpec((BM, 48), lambda i: (i, 0)),
            pl.BlockSpec((BM, 1), lambda i: (i, 0)),
            pl.BlockSpec((BM, 1), lambda i: (i, 0)),
            pl.BlockSpec((1, 32), lambda i: (0, 0)),
            pl.BlockSpec((32, 2), lambda i: (0, 0)),
            pl.BlockSpec((1, 2), lambda i: (0, 0)),
        ],
        out_specs=[
            pl.BlockSpec((1, 32), lambda i: (0, 0)),
            pl.BlockSpec((1, 2), lambda i: (0, 0)),
        ],
        out_shape=[
            jax.ShapeDtypeStruct((1, 32), jnp.float32),
            jax.ShapeDtypeStruct((1, 2), jnp.float32),
        ],
    )(num, hg, es, ed, bg, Wf, bf)
    return out


_scatter_2x256 = _make_sc_scatter(2, 256, jnp.bfloat16)
_scatter_1x128 = _make_sc_scatter(1, 128, jnp.bfloat16)
_scatter_1x32 = _make_sc_scatter(1, 32, jnp.bfloat16)


def kernel(x, edge_index, W1, b1, g1, be1, W2, b2, g2, be2, W3, b3, g3, be3,
           Wg, a_s, a_d, bg, Wf, bf):
    f32 = jnp.float32
    xp = jnp.pad(x, ((0, PN - N_NODES), (0, 0)))
    pad = jnp.full((EP - E_RAW,), PAD_NODE, jnp.int32)
    src2d = jnp.concatenate([edge_index[0], pad]).reshape(EP // 128, 128)
    dst2d = jnp.concatenate([edge_index[1], pad]).reshape(EP // 128, 128)
    bf16 = jnp.bfloat16
    z256b = jnp.zeros((128, 256), bf16)
    z128b = jnp.zeros((128, 128), bf16)
    z32b = jnp.zeros((128, 32), bf16)
    z48 = jnp.zeros((128, 48), f32)

    degp = _sc_degree(dst2d)
    dinv = _reduce_parts(degp, is_deg=True)

    h1s, h1sb = _mm1(xp, W1, dinv)
    S1 = _scatter_2x256(h1sb, src2d, dst2d, z256b)
    h2s, h2sb = _post_mm(2, 256, 128, S1, h1s, dinv,
                         b1.reshape(2, 256), g1.reshape(2, 256),
                         be1.reshape(2, 256),
                         W2.reshape(2, 256, 128))
    S2 = _scatter_1x128(h2sb, src2d, dst2d, z128b)
    h3s, h3sb = _post_mm(1, 128, 32, S2, h2s, dinv,
                         b2.reshape(1, 128), g2.reshape(1, 128),
                         be2.reshape(1, 128),
                         W3.reshape(1, 128, 32))
    S3 = _scatter_1x32(h3sb, src2d, dst2d, z32b)
    hg, es, ed = _gatprep(S3, h3s, dinv,
                          b3.reshape(1, 32), g3.reshape(1, 32),
                          be3.reshape(1, 32), Wg,
                          a_s.reshape(32, 1), a_d.reshape(32, 1))
    num = _sc_gat(hg, es.reshape(PR, 128), ed.reshape(PR, 128),
                  src2d, dst2d, z48)
    return _gatfinal(num, hg, es, ed, bg.reshape(1, 32), Wf,
                     bf.reshape(1, 2))


# speedup vs baseline: 1.1489x; 1.0161x over previous
import functools

import jax
import jax.numpy as jnp
from jax import lax
from jax.experimental import pallas as pl
from jax.experimental.pallas import tpu as pltpu
from jax.experimental.pallas import tpu_sc as plsc

N_NODES = 10000
IN_DIM = 2048
PN = 10240
E_RAW = 160000
EP = 163840
NC, NS = 2, 16
GP = EP // (NC * NS * 128)
GP0, GP1 = 62, 18
GPMX = max(GP0, GP1)
RPT = PN // NS
PAD_NODE = PN - 1
BM = 256
NB = PN // BM


def _sc_mesh():
    return plsc.VectorSubcoreMesh(
        core_axis_name="c", subcore_axis_name="s", num_cores=NC, num_subcores=NS
    )


PR = PN // 128


def _split_idx(i16):
    return [lax.shift_right_logical(i16, 7), jnp.bitwise_and(i16, 127)]


@functools.partial(
    pl.kernel,
    out_type=jax.ShapeDtypeStruct((NC, NS, PR, 128), jnp.float32),
    mesh=_sc_mesh(),
    compiler_params=pltpu.CompilerParams(
        needs_layout_passes=False, use_tc_tiling_on_sc=False),
    scratch_types=[
        pltpu.VMEM((GPMX, 128), jnp.int32),
        pltpu.VMEM((PR, 128), jnp.float32),
    ],
    name="sc_degree",
)
def _sc_degree(dst_hbm, out_hbm, dst_v, part_v):
    cid = lax.axis_index("c")
    sid = lax.axis_index("s")
    cnt = jnp.where(cid == 0, GP0, GP1)
    base = jnp.where(cid == 0, sid * GP0, NS * GP0 + sid * GP1)
    pltpu.sync_copy(dst_hbm.at[pl.ds(base, GPMX)], dst_v)
    zeros = jnp.zeros((16,), jnp.float32)

    @pl.loop(0, PR)
    def _(i):
        @pl.loop(0, 8)
        def _(j):
            part_v[i, pl.ds(j * 16, 16)] = zeros

    ones = jnp.ones((16,), jnp.float32)

    @pl.loop(0, cnt)
    def _(g):
        @pl.loop(0, 8)
        def _(j):
            d16 = dst_v[g, pl.ds(j * 16, 16)]
            plsc.addupdate_scatter(part_v, _split_idx(d16), ones)

    pltpu.sync_copy(part_v, out_hbm.at[cid, sid])


def _make_sc_scatter(C, Fc, dtype=jnp.float32):
    @functools.partial(
        pl.kernel,
        out_type=jax.ShapeDtypeStruct((NC, C, PN, Fc), dtype),
        mesh=_sc_mesh(),
        compiler_params=pltpu.CompilerParams(
            needs_layout_passes=False, use_tc_tiling_on_sc=False),
        scratch_types=[
            pltpu.VMEM((GPMX, 128), jnp.int32),
            pltpu.VMEM((GPMX, 128), jnp.int32),
            pltpu.VMEM((128, Fc), dtype),
            pltpu.VMEM((128, Fc), dtype),
            pltpu.VMEM_SHARED((PN, Fc), dtype),
            pltpu.SemaphoreType.DMA,
            pltpu.SemaphoreType.DMA,
            pltpu.SemaphoreType.DMA,
            pltpu.SemaphoreType.DMA,
        ],
        name=f"sc_scatter_{C}x{Fc}",
    )
    def k(h_hbm, src_hbm, dst_hbm, z_hbm, out_hbm,
          src_v, dst_v, r0, r1, acc_sh, sg0, sg1, ss0, ss1):
        cid = lax.axis_index("c")
        sid = lax.axis_index("s")
        cnt = jnp.where(cid == 0, GP0, GP1)
        base = jnp.where(cid == 0, sid * GP0, NS * GP0 + sid * GP1)
        pltpu.sync_copy(src_hbm.at[pl.ds(base, GPMX)], src_v)
        pltpu.sync_copy(dst_hbm.at[pl.ds(base, GPMX)], dst_v)
        for c in range(C):
            for z in range(RPT // 128):
                pltpu.sync_copy(
                    z_hbm, acc_sh.at[pl.ds(sid * RPT + z * 128, 128)]
                )
            plsc.subcore_barrier()
            pltpu.async_copy(h_hbm.at[c].at[src_v.at[0]], r0, sg0)

            @pl.loop(0, cnt // 2)
            def _(t):
                g = t * 2
                pltpu.make_async_copy(
                    h_hbm.at[c].at[src_v.at[g]], r0, sg0).wait()
                pltpu.async_copy(h_hbm.at[c].at[src_v.at[g + 1]], r1, sg1)
                pltpu.async_copy(r0, acc_sh.at[dst_v.at[g]], ss0, add=True)
                pltpu.make_async_copy(
                    h_hbm.at[c].at[src_v.at[g + 1]], r1, sg1).wait()
                pltpu.make_async_copy(r0, acc_sh.at[dst_v.at[g]], ss0).wait()

                @pl.when(t < cnt // 2 - 1)
                def _():
                    pltpu.async_copy(
                        h_hbm.at[c].at[src_v.at[g + 2]], r0, sg0)

                pltpu.async_copy(r1, acc_sh.at[dst_v.at[g + 1]], ss1, add=True)
                pltpu.make_async_copy(
                    r1, acc_sh.at[dst_v.at[g + 1]], ss1).wait()

            plsc.subcore_barrier()
            pltpu.sync_copy(
                acc_sh.at[pl.ds(sid * RPT, RPT)],
                out_hbm.at[cid, c, pl.ds(sid * RPT, RPT)],
            )
            if c < C - 1:
                plsc.subcore_barrier()

    return k


@functools.partial(
    pl.kernel,
    out_type=jax.ShapeDtypeStruct((NC, PN, 48), jnp.float32),
    mesh=_sc_mesh(),
    compiler_params=pltpu.CompilerParams(
        needs_layout_passes=False, use_tc_tiling_on_sc=False),
    scratch_types=[
        pltpu.VMEM((GPMX, 128), jnp.int32),
        pltpu.VMEM((GPMX, 128), jnp.int32),
        pltpu.VMEM((PR, 128), jnp.float32),
        pltpu.VMEM((PR, 128), jnp.float32),
        pltpu.VMEM((128,), jnp.float32),
        pltpu.VMEM((128, 48), jnp.float32),
        pltpu.VMEM((128, 48), jnp.float32),
        pltpu.VMEM_SHARED((PN, 48), jnp.float32),
        pltpu.SemaphoreType.DMA,
        pltpu.SemaphoreType.DMA,
        pltpu.SemaphoreType.DMA,
        pltpu.SemaphoreType.DMA,
    ],
    name="sc_gat",
)
def _sc_gat(hg_hbm, es_hbm, ed_hbm, src_hbm, dst_hbm, z_hbm,
            num_hbm,
            src_v, dst_v, es_v, ed_v, w_v, r0, r1, acc_sh,
            sg0, sg1, ss0, ss1):
    cid = lax.axis_index("c")
    sid = lax.axis_index("s")
    cnt = jnp.where(cid == 0, GP0, GP1)
    base = jnp.where(cid == 0, sid * GP0, NS * GP0 + sid * GP1)
    pltpu.sync_copy(src_hbm.at[pl.ds(base, GPMX)], src_v)
    pltpu.sync_copy(dst_hbm.at[pl.ds(base, GPMX)], dst_v)
    pltpu.sync_copy(es_hbm, es_v)
    pltpu.sync_copy(ed_hbm, ed_v)
    for z in range(RPT // 128):
        pltpu.sync_copy(z_hbm, acc_sh.at[pl.ds(sid * RPT + z * 128, 128)])
    plsc.subcore_barrier()

    onehot = (lax.iota(jnp.int32, 16) == 0).astype(jnp.float32)

    def _weigh(g, rows_v):
        @pl.loop(0, 8)
        def _(j):
            s16 = src_v[g, pl.ds(j * 16, 16)]
            d16 = dst_v[g, pl.ds(j * 16, 16)]
            z16 = (plsc.load_gather(es_v, _split_idx(s16))
                   + plsc.load_gather(ed_v, _split_idx(d16)))
            w16 = jnp.exp(jnp.maximum(z16, 0.2 * z16))
            w_v[pl.ds(j * 16, 16)] = w16

        @pl.loop(0, 8)
        def _(r):
            w16 = w_v[pl.ds(r * 16, 16)]
            for k in range(16):
                w_s = w16[k]
                row = r * 16 + k
                rows_v[row, pl.ds(0, 16)] = rows_v[row, pl.ds(0, 16)] * w_s
                rows_v[row, pl.ds(16, 16)] = rows_v[row, pl.ds(16, 16)] * w_s
                rows_v[row, pl.ds(32, 16)] = onehot * w_s

    pltpu.async_copy(hg_hbm.at[src_v.at[0]], r0, sg0)

    @pl.loop(0, cnt // 2)
    def _(t):
        g = t * 2
        pltpu.make_async_copy(hg_hbm.at[src_v.at[g]], r0, sg0).wait()
        pltpu.async_copy(hg_hbm.at[src_v.at[g + 1]], r1, sg1)
        _weigh(g, r0)
        pltpu.async_copy(r0, acc_sh.at[dst_v.at[g]], ss0, add=True)
        pltpu.make_async_copy(hg_hbm.at[src_v.at[g + 1]], r1, sg1).wait()
        pltpu.make_async_copy(r0, acc_sh.at[dst_v.at[g]], ss0).wait()

        @pl.when(t < cnt // 2 - 1)
        def _():
            pltpu.async_copy(hg_hbm.at[src_v.at[g + 2]], r0, sg0)

        _weigh(g + 1, r1)
        pltpu.async_copy(r1, acc_sh.at[dst_v.at[g + 1]], ss1, add=True)
        pltpu.make_async_copy(r1, acc_sh.at[dst_v.at[g + 1]], ss1).wait()

    plsc.subcore_barrier()
    pltpu.sync_copy(
        acc_sh.at[pl.ds(sid * RPT, RPT)],
        num_hbm.at[cid, pl.ds(sid * RPT, RPT)],
    )


def _rdeg_body(p_ref, out_ref):
    deg = jnp.sum(p_ref[...], axis=(0, 1)) + 1.0
    out_ref[...] = lax.rsqrt(deg)


def _rden_body(p_ref, out_ref):
    out_ref[...] = jnp.sum(p_ref[...], axis=(0, 1))


def _reduce_parts(parts, is_deg):
    out2d = pl.pallas_call(
        _rdeg_body if is_deg else _rden_body,
        out_shape=jax.ShapeDtypeStruct((PR, 128), jnp.float32),
    )(parts)
    return out2d.reshape(PN, 1)


def _mm1_body(x_ref, w_ref, dinv_ref, out_ref, outb_ref):
    h = jnp.dot(x_ref[...], w_ref[...], preferred_element_type=jnp.float32)
    hs = h * dinv_ref[...]
    for c in range(2):
        blk = hs[:, c * 256:(c + 1) * 256]
        out_ref[c, :, :] = blk
        outb_ref[c, :, :] = blk.astype(jnp.bfloat16)


def _mm1(xp, W1, dinv):
    return pl.pallas_call(
        _mm1_body,
        grid=(NB,),
        in_specs=[
            pl.BlockSpec((BM, IN_DIM), lambda i: (i, 0)),
            pl.BlockSpec((IN_DIM, 512), lambda i: (0, 0)),
            pl.BlockSpec((BM, 1), lambda i: (i, 0)),
        ],
        out_specs=[
            pl.BlockSpec((2, BM, 256), lambda i: (0, i, 0)),
            pl.BlockSpec((2, BM, 256), lambda i: (0, i, 0)),
        ],
        out_shape=[
            jax.ShapeDtypeStruct((2, PN, 256), jnp.float32),
            jax.ShapeDtypeStruct((2, PN, 256), jnp.bfloat16),
        ],
    )(xp, W1, dinv)


def _post_mm_body(C_in, F_out, s_ref, hs_ref, dinv_ref, b_ref, g_ref,
                  be_ref, w_ref, out_ref, outb_ref):
    dv = dinv_ref[...]
    sagg = (s_ref[0].astype(jnp.float32) + s_ref[1].astype(jnp.float32))
    y = (sagg + hs_ref[...]) * dv[None]
    y = y + b_ref[...][:, None, :]
    mu = jnp.mean(y, axis=(0, 2), keepdims=True)
    var = jnp.mean((y - mu) ** 2, axis=(0, 2), keepdims=True)
    yn = (y - mu) * lax.rsqrt(var + 1e-5)
    yn = yn * g_ref[...][:, None, :] + be_ref[...][:, None, :]
    xr = jnp.maximum(yn, 0.0)
    acc = jnp.zeros((BM, F_out), jnp.float32)
    for c in range(C_in):
        acc = acc + jnp.dot(xr[c], w_ref[c], preferred_element_type=jnp.float32)
    res = acc * dv
    out_ref[0, :, :] = res
    outb_ref[0, :, :] = res.astype(jnp.bfloat16)


def _post_mm(C_in, Fc_in, F_out, S, hs, dinv, b, g, be, Wc):
    body = functools.partial(_post_mm_body, C_in, F_out)
    return pl.pallas_call(
        body,
        grid=(NB,),
        in_specs=[
            pl.BlockSpec((NC, C_in, BM, Fc_in), lambda i: (0, 0, i, 0)),
            pl.BlockSpec((C_in, BM, Fc_in), lambda i: (0, i, 0)),
            pl.BlockSpec((BM, 1), lambda i: (i, 0)),
            pl.BlockSpec((C_in, Fc_in), lambda i: (0, 0)),
            pl.BlockSpec((C_in, Fc_in), lambda i: (0, 0)),
            pl.BlockSpec((C_in, Fc_in), lambda i: (0, 0)),
            pl.BlockSpec((C_in, Fc_in, F_out), lambda i: (0, 0, 0)),
        ],
        out_specs=[
            pl.BlockSpec((1, BM, F_out), lambda i: (0, i, 0)),
            pl.BlockSpec((1, BM, F_out), lambda i: (0, i, 0)),
        ],
        out_shape=[
            jax.ShapeDtypeStruct((1, PN, F_out), jnp.float32),
            jax.ShapeDtypeStruct((1, PN, F_out), jnp.bfloat16),
        ],
    )(S, hs, dinv, b, g, be, Wc)


def _gatprep_body(s_ref, hs_ref, dinv_ref, b_ref, g_ref, be_ref,
                  wg_ref, as_ref, ad_ref, hg_ref, es_ref, ed_ref):
    dv = dinv_ref[...]
    y = (s_ref[0, 0].astype(jnp.float32) + s_ref[1, 0].astype(jnp.float32)
         + hs_ref[0]) * dv + b_ref[...]
    mu = jnp.mean(y, axis=1, keepdims=True)
    var = jnp.mean((y - mu) ** 2, axis=1, keepdims=True)
    yn = (y - mu) * lax.rsqrt(var + 1e-5) * g_ref[...] + be_ref[...]
    x4 = jnp.maximum(yn, 0.0)
    hg = jnp.dot(x4, wg_ref[...], preferred_element_type=jnp.float32)
    hg_ref[...] = jnp.concatenate(
        [hg, jnp.zeros((BM, 16), jnp.float32)], axis=1)
    es_ref[...] = jnp.dot(hg, as_ref[...], preferred_element_type=jnp.float32)
    ed_ref[...] = jnp.dot(hg, ad_ref[...], preferred_element_type=jnp.float32)


def _gatprep(S3, h3s, dinv, b3, g3, be3, Wg, a_s, a_d):
    return pl.pallas_call(
        _gatprep_body,
        grid=(NB,),
        in_specs=[
            pl.BlockSpec((NC, 1, BM, 32), lambda i: (0, 0, i, 0)),
            pl.BlockSpec((1, BM, 32), lambda i: (0, i, 0)),
            pl.BlockSpec((BM, 1), lambda i: (i, 0)),
            pl.BlockSpec((1, 32), lambda i: (0, 0)),
            pl.BlockSpec((1, 32), lambda i: (0, 0)),
            pl.BlockSpec((1, 32), lambda i: (0, 0)),
            pl.BlockSpec((32, 32), lambda i: (0, 0)),
            pl.BlockSpec((32, 1), lambda i: (0, 0)),
            pl.BlockSpec((32, 1), lambda i: (0, 0)),
        ],
        out_specs=[
            pl.BlockSpec((BM, 48), lambda i: (i, 0)),
            pl.BlockSpec((BM, 1), lambda i: (i, 0)),
            pl.BlockSpec((BM, 1), lambda i: (i, 0)),
        ],
        out_shape=[
            jax.ShapeDtypeStruct((PN, 48), jnp.float32),
            jax.ShapeDtypeStruct((PN, 1), jnp.float32),
            jax.ShapeDtypeStruct((PN, 1), jnp.float32),
        ],
    )(S3, h3s, dinv, b3, g3, be3, Wg, a_s, a_d)


def _gatfinal_body(num_ref, hg_ref, es_ref, ed_ref, bg_ref,
                   wf_ref, bf_ref, acc_ref, out_ref):
    i = pl.program_id(0)
    z = es_ref[...] + ed_ref[...]
    wself = jnp.exp(jnp.maximum(z, 0.2 * z))
    hg = hg_ref[...][:, :32]
    nd = num_ref[0] + num_ref[1]
    num = nd[:, :32] + wself * hg
    den = nd[:, 32:33] + wself + 1e-16
    hgat = jnp.maximum(num / den + bg_ref[...], 0.0)
    rows = lax.broadcasted_iota(jnp.int32, (BM, 1), 0) + i * BM
    hgat = jnp.where(rows < N_NODES, hgat, 0.0)
    part = jnp.sum(hgat, axis=0, keepdims=True)

    @pl.when(i == 0)
    def _():
        acc_ref[...] = jnp.zeros_like(acc_ref)

    acc_ref[...] += part

    @pl.when(i == NB - 1)
    def _():
        pooled = acc_ref[...] * (1.0 / N_NODES)
        out_ref[...] = (
            jnp.dot(pooled, wf_ref[...], preferred_element_type=jnp.float32)
            + bf_ref[...]
        )


def _gatfinal(num, hg, es, ed, bg, Wf, bf):
    _, out = pl.pallas_call(
        _gatfinal_body,
        grid=(NB,),
        in_specs=[
            pl.BlockSpec((NC, BM, 48), lambda i: (0, i, 0)),
            pl.BlockSpec((BM, 48), lambda i: (i, 0)),
            pl.BlockSpec((BM, 1), lambda i: (i, 0)),
            pl.BlockSpec((BM, 1), lambda i: (i, 0)),
            pl.BlockSpec((1, 32), lambda i: (0, 0)),
            pl.BlockSpec((32, 2), lambda i: (0, 0)),
            pl.BlockSpec((1, 2), lambda i: (0, 0)),
        ],
        out_specs=[
            pl.BlockSpec((1, 32), lambda i: (0, 0)),
            pl.BlockSpec((1, 2), lambda i: (0, 0)),
        ],
        out_shape=[
            jax.ShapeDtypeStruct((1, 32), jnp.float32),
            jax.ShapeDtypeStruct((1, 2), jnp.float32),
        ],
    )(num, hg, es, ed, bg, Wf, bf)
    return out


_scatter_2x256 = _make_sc_scatter(2, 256, jnp.bfloat16)
_scatter_1x128 = _make_sc_scatter(1, 128, jnp.bfloat16)
_scatter_1x32 = _make_sc_scatter(1, 32, jnp.bfloat16)


def kernel(x, edge_index, W1, b1, g1, be1, W2, b2, g2, be2, W3, b3, g3, be3,
           Wg, a_s, a_d, bg, Wf, bf):
    f32 = jnp.float32
    xp = jnp.pad(x, ((0, PN - N_NODES), (0, 0)))
    pad = jnp.full((EP - E_RAW,), PAD_NODE, jnp.int32)
    src2d = jnp.concatenate([edge_index[0], pad]).reshape(EP // 128, 128)
    dst2d = jnp.concatenate([edge_index[1], pad]).reshape(EP // 128, 128)
    bf16 = jnp.bfloat16
    z256b = jnp.zeros((128, 256), bf16)
    z128b = jnp.zeros((128, 128), bf16)
    z32b = jnp.zeros((128, 32), bf16)
    z48 = jnp.zeros((128, 48), f32)

    degp = _sc_degree(dst2d)
    dinv = _reduce_parts(degp, is_deg=True)

    h1s, h1sb = _mm1(xp, W1, dinv)
    S1 = _scatter_2x256(h1sb, src2d, dst2d, z256b)
    h2s, h2sb = _post_mm(2, 256, 128, S1, h1s, dinv,
                         b1.reshape(2, 256), g1.reshape(2, 256),
                         be1.reshape(2, 256),
                         W2.reshape(2, 256, 128))
    S2 = _scatter_1x128(h2sb, src2d, dst2d, z128b)
    h3s, h3sb = _post_mm(1, 128, 32, S2, h2s, dinv,
                         b2.reshape(1, 128), g2.reshape(1, 128),
                         be2.reshape(1, 128),
                         W3.reshape(1, 128, 32))
    S3 = _scatter_1x32(h3sb, src2d, dst2d, z32b)
    hg, es, ed = _gatprep(S3, h3s, dinv,
                          b3.reshape(1, 32), g3.reshape(1, 32),
                          be3.reshape(1, 32), Wg,
                          a_s.reshape(32, 1), a_d.reshape(32, 1))
    num = _sc_gat(hg, es.reshape(PR, 128), ed.reshape(PR, 128),
                  src2d, dst2d, z48)
    return _gatfinal(num, hg, es, ed, bg.reshape(1, 32), Wf,
                     bf.reshape(1, 2))
